# baseline (speedup 1.0000x reference)
import functools

import jax
import jax.numpy as jnp
from jax import lax
from jax.experimental import pallas as pl
from jax.experimental.pallas import tpu as pltpu
from jax.experimental.pallas import tpu_sc as plsc

BATCH = 4096
SEQ = 200
EMBED_DIM = 64
B = BATCH * SEQ
NW = 32
B_PER_W = B // NW
CHUNK = 400
NBUF = 4
N_CHUNKS = B_PER_W // CHUNK
N_GROUPS = N_CHUNKS // NBUF


def _make_emb_kernel():
    mesh = plsc.VectorSubcoreMesh(core_axis_name="c", subcore_axis_name="s")

    @functools.partial(
        pl.kernel,
        mesh=mesh,
        out_type=jax.ShapeDtypeStruct((B, EMBED_DIM), jnp.float32),
        scratch_types=[
            pltpu.VMEM((B_PER_W,), jnp.int32),
            pltpu.VMEM((NBUF, CHUNK, EMBED_DIM), jnp.float32),
        ] + [pltpu.SemaphoreType.DMA] * (2 * NBUF),
        compiler_params=pltpu.CompilerParams(use_tc_tiling_on_sc=False),
    )
    def emb(idx_hbm, table_hbm, out_hbm, idx_v, rows_v, *sems):
        gsem = sems[:NBUF]
        wsem = sems[NBUF:]
        wid = lax.axis_index("s") * 2 + lax.axis_index("c")
        base = wid * B_PER_W
        pltpu.sync_copy(idx_hbm.at[pl.ds(base, B_PER_W)], idx_v)

        def gather_start(b, i):
            pltpu.async_copy(
                table_hbm.at[idx_v.at[pl.ds(i * CHUNK, CHUNK)]],
                rows_v.at[b], gsem[b])

        def gather_wait(b, i):
            pltpu.make_async_copy(
                table_hbm.at[idx_v.at[pl.ds(i * CHUNK, CHUNK)]],
                rows_v.at[b], gsem[b]).wait()

        def write_start(b, i):
            pltpu.async_copy(
                rows_v.at[b],
                out_hbm.at[pl.ds(base + i * CHUNK, CHUNK)], wsem[b])

        def write_wait(b, i):
            pltpu.make_async_copy(
                rows_v.at[b],
                out_hbm.at[pl.ds(base + i * CHUNK, CHUNK)], wsem[b]).wait()

        for b in range(NBUF):
            gather_start(b, b)

        def group(g, carry):
            j = g * NBUF
            for b in range(NBUF):
                gather_wait(b, j + b)
                write_start(b, j + b)
            for b in range(NBUF):
                write_wait(b, j + b)
                gather_start(b, j + NBUF + b)
            return carry

        lax.fori_loop(0, N_GROUPS - 1, group, 0)

        j = (N_GROUPS - 1) * NBUF
        for b in range(NBUF):
            gather_wait(b, j + b)
            write_start(b, j + b)
        for b in range(NBUF):
            write_wait(b, j + b)

    return emb


_emb = _make_emb_kernel()


def kernel(sentence, table):
    idx = jnp.reshape(sentence, (B,)).astype(jnp.int32)
    out = _emb(idx, table)
    return jnp.reshape(out, (BATCH, SEQ, EMBED_DIM))
